# baseline (speedup 1.0000x reference)
import functools

import jax
import jax.numpy as jnp
from jax import lax
from jax.experimental import pallas as pl
from jax.experimental.pallas import tpu as pltpu
from jax.experimental.pallas import tpu_sc as plsc

_L = 16
_NBINS = 256
_N = 32768
_ROWS = 64
_NW = 32
_ROWS_PER_W = _ROWS // _NW


def _zero_hist(hist_ref):
    z = jnp.zeros((_L,), jnp.int32)

    def body(i, c):
        hist_ref[pl.ds(i * _L, _L)] = z
        return c

    lax.fori_loop(0, (_NBINS * _L) // _L, body, 0)


def _hist_pass(row_ref, hist_ref, shift, prefix, lvl):
    ones = jnp.ones((_L,), jnp.int32)
    laneoff = lax.iota(jnp.int32, _L) * _NBINS

    def body(i, c):
        b = row_ref[pl.ds(i * _L, _L)] & jnp.int32(0x7FFFFFFF)
        dig = lax.shift_right_logical(b, shift) & jnp.int32(0xFF)
        idx = laneoff + dig
        if lvl == 0:
            plsc.addupdate_scatter(hist_ref, [idx], ones)
        else:
            m = lax.shift_right_logical(b, shift + 8) == prefix
            plsc.addupdate_scatter(hist_ref, [idx], ones, mask=m)
        return c

    lax.fori_loop(0, _N // _L, body, 0)


def _collapse(hist_ref, cnt_ref):
    def body(g, c):
        acc = hist_ref[pl.ds(g * _L, _L)]
        for lane in range(1, _L):
            acc = acc + hist_ref[pl.ds(lane * _NBINS + g * _L, _L)]
        cnt_ref[pl.ds(g * _L, _L)] = acc
        return c

    lax.fori_loop(0, _NBINS // _L, body, 0)


def _gather16(x, idx):
    dn = lax.GatherDimensionNumbers(
        offset_dims=(), collapsed_slice_dims=(0,), start_index_map=(0,))
    return lax.gather(
        x, idx.reshape(_L, 1), dn, slice_sizes=(1,),
        mode=lax.GatherScatterMode.PROMISE_IN_BOUNDS)


def _cumsum16(x):
    lanes = lax.iota(jnp.int32, _L)
    s = x
    for d in (1, 2, 4, 8):
        sh = _gather16(s, jnp.maximum(lanes - d, 0))
        s = s + jnp.where(lanes >= d, sh, 0)
    return s


def _top_lane():
    return jnp.full((_L,), _L - 1, jnp.int32)


def _select_digit(cnt_ref, j):
    zero = jnp.zeros((_L,), jnp.int32)

    def body(i, carry):
        above, d_star, j_next, done = carry
        g = 15 - i
        h = cnt_ref[pl.ds(g * _L, _L)]
        cs = _cumsum16(h)
        gsum = _gather16(cs, _top_lane())
        s_vec = above + gsum - cs + h
        m = s_vec >= j
        pc = _gather16(_cumsum16(m.astype(jnp.int32)), _top_lane())
        hit = jnp.logical_and(pc > 0, done == 0)
        lanepos = jnp.maximum(pc - 1, 0)
        cs_at = _gather16(cs, lanepos)
        d_star = jnp.where(hit, g * _L + lanepos, d_star)
        j_next = jnp.where(hit, j - (above + gsum - cs_at), j_next)
        done = jnp.where(pc > 0, 1, done)
        above = jnp.where(done > 0, above, above + gsum)
        return above, d_star, j_next, done

    _, d_star, j_next, _ = lax.fori_loop(
        0, _NBINS // _L, body, (zero, zero, j, zero))
    return d_star, j_next


def _process_row(row_ref, hist_ref, cnt_ref, kk):
    prefix = jnp.zeros((_L,), jnp.int32)
    j = kk
    for lvl in range(4):
        shift = 24 - 8 * lvl
        _zero_hist(hist_ref)
        _hist_pass(row_ref, hist_ref, shift, prefix, lvl)
        _collapse(hist_ref, cnt_ref)
        d, j = _select_digit(cnt_ref, j)
        prefix = prefix * 256 + d
    thr = prefix

    def apply_body(i, c):
        v = row_ref[pl.ds(i * _L, _L)]
        b = v & jnp.int32(0x7FFFFFFF)
        row_ref[pl.ds(i * _L, _L)] = jnp.where(b >= thr, v, 0)
        return c

    lax.fori_loop(0, _N // _L, apply_body, 0)


_MESH = plsc.VectorSubcoreMesh(core_axis_name="c", subcore_axis_name="s")


@functools.partial(
    pl.kernel,
    mesh=_MESH,
    compiler_params=pltpu.CompilerParams(needs_layout_passes=False),
    out_type=jax.ShapeDtypeStruct((_ROWS * _N,), jnp.int32),
    scratch_types=[
        pltpu.VMEM((_N,), jnp.int32),
        pltpu.VMEM((_NBINS * _L,), jnp.int32),
        pltpu.VMEM((_NBINS,), jnp.int32),
        pltpu.VMEM((_L,), jnp.int32),
    ],
)
def _sc_topk(scores_hbm, kvec_hbm, out_hbm, row_v, hist_v, cnt_v, k_v):
    wid = lax.axis_index("s") * 2 + lax.axis_index("c")
    pltpu.sync_copy(kvec_hbm, k_v)
    kk = k_v[...]
    for r in range(_ROWS_PER_W):
        row = wid * _ROWS_PER_W + r
        base = row * _N
        pltpu.sync_copy(scores_hbm.at[pl.ds(base, _N)], row_v)
        _process_row(row_v, hist_v, cnt_v, kk)
        pltpu.sync_copy(row_v, out_hbm.at[pl.ds(base, _N)])


def kernel(scores, k):
    rows, cols = scores.shape
    kvec = jnp.full((_L,), k, jnp.int32)
    bits = lax.bitcast_convert_type(scores.reshape(-1), jnp.int32)
    out = _sc_topk(bits, kvec)
    return lax.bitcast_convert_type(out, jnp.float32).reshape(rows, cols)


# speedup vs baseline: 1.1670x; 1.1670x over previous
import functools

import jax
import jax.numpy as jnp
from jax import lax
from jax.experimental import pallas as pl
from jax.experimental.pallas import tpu as pltpu
from jax.experimental.pallas import tpu_sc as plsc

_L = 16
_NBINS = 256
_N = 32768
_ROWS = 64
_NW = 32
_ROWS_PER_W = _ROWS // _NW


_U = 8


def _zero_hist(hist_ref):
    z = jnp.zeros((_L,), jnp.int32)

    def body(i, c):
        for u in range(_U):
            hist_ref[pl.ds((i * _U + u) * _L, _L)] = z
        return c

    lax.fori_loop(0, _NBINS // _U, body, 0)


def _hist_pass(row_ref, hist_ref, shift, prefix, lvl):
    ones = jnp.ones((_L,), jnp.int32)
    laneoff = lax.iota(jnp.int32, _L) * _NBINS

    def body(i, c):
        for u in range(_U):
            b = row_ref[pl.ds((i * _U + u) * _L, _L)] & jnp.int32(0x7FFFFFFF)
            dig = lax.shift_right_logical(b, shift) & jnp.int32(0xFF)
            idx = laneoff + dig
            if lvl == 0:
                plsc.addupdate_scatter(hist_ref, [idx], ones)
            else:
                m = lax.shift_right_logical(b, shift + 8) == prefix
                plsc.addupdate_scatter(hist_ref, [idx], ones, mask=m)
        return c

    lax.fori_loop(0, _N // (_L * _U), body, 0)


def _collapse(hist_ref, cnt_ref):
    def body(g, c):
        acc = hist_ref[pl.ds(g * _L, _L)]
        for lane in range(1, _L):
            acc = acc + hist_ref[pl.ds(lane * _NBINS + g * _L, _L)]
        cnt_ref[pl.ds(g * _L, _L)] = acc
        return c

    lax.fori_loop(0, _NBINS // _L, body, 0)


def _gather16(x, idx):
    dn = lax.GatherDimensionNumbers(
        offset_dims=(), collapsed_slice_dims=(0,), start_index_map=(0,))
    return lax.gather(
        x, idx.reshape(_L, 1), dn, slice_sizes=(1,),
        mode=lax.GatherScatterMode.PROMISE_IN_BOUNDS)


def _cumsum16(x):
    lanes = lax.iota(jnp.int32, _L)
    s = x
    for d in (1, 2, 4, 8):
        sh = _gather16(s, jnp.maximum(lanes - d, 0))
        s = s + jnp.where(lanes >= d, sh, 0)
    return s


def _top_lane():
    return jnp.full((_L,), _L - 1, jnp.int32)


def _select_digit(cnt_ref, j):
    zero = jnp.zeros((_L,), jnp.int32)

    def body(i, carry):
        above, d_star, j_next, done = carry
        g = 15 - i
        h = cnt_ref[pl.ds(g * _L, _L)]
        cs = _cumsum16(h)
        gsum = _gather16(cs, _top_lane())
        s_vec = above + gsum - cs + h
        m = s_vec >= j
        pc = _gather16(_cumsum16(m.astype(jnp.int32)), _top_lane())
        hit = jnp.logical_and(pc > 0, done == 0)
        lanepos = jnp.maximum(pc - 1, 0)
        cs_at = _gather16(cs, lanepos)
        d_star = jnp.where(hit, g * _L + lanepos, d_star)
        j_next = jnp.where(hit, j - (above + gsum - cs_at), j_next)
        done = jnp.where(pc > 0, 1, done)
        above = jnp.where(done > 0, above, above + gsum)
        return above, d_star, j_next, done

    _, d_star, j_next, _ = lax.fori_loop(
        0, _NBINS // _L, body, (zero, zero, j, zero))
    return d_star, j_next


def _process_row(row_ref, hist_ref, cnt_ref, kk):
    prefix = jnp.zeros((_L,), jnp.int32)
    j = kk
    for lvl in range(4):
        shift = 24 - 8 * lvl
        _zero_hist(hist_ref)
        _hist_pass(row_ref, hist_ref, shift, prefix, lvl)
        _collapse(hist_ref, cnt_ref)
        d, j = _select_digit(cnt_ref, j)
        prefix = prefix * 256 + d
    thr = prefix

    def apply_body(i, c):
        for u in range(_U):
            ds = pl.ds((i * _U + u) * _L, _L)
            v = row_ref[ds]
            b = v & jnp.int32(0x7FFFFFFF)
            row_ref[ds] = jnp.where(b >= thr, v, 0)
        return c

    lax.fori_loop(0, _N // (_L * _U), apply_body, 0)


_MESH = plsc.VectorSubcoreMesh(core_axis_name="c", subcore_axis_name="s")


@functools.partial(
    pl.kernel,
    mesh=_MESH,
    compiler_params=pltpu.CompilerParams(needs_layout_passes=False),
    out_type=jax.ShapeDtypeStruct((_ROWS * _N,), jnp.int32),
    scratch_types=[
        pltpu.VMEM((_N,), jnp.int32),
        pltpu.VMEM((_NBINS * _L,), jnp.int32),
        pltpu.VMEM((_NBINS,), jnp.int32),
        pltpu.VMEM((_L,), jnp.int32),
    ],
)
def _sc_topk(scores_hbm, kvec_hbm, out_hbm, row_v, hist_v, cnt_v, k_v):
    wid = lax.axis_index("s") * 2 + lax.axis_index("c")
    pltpu.sync_copy(kvec_hbm, k_v)
    kk = k_v[...]
    for r in range(_ROWS_PER_W):
        row = wid * _ROWS_PER_W + r
        base = row * _N
        pltpu.sync_copy(scores_hbm.at[pl.ds(base, _N)], row_v)
        _process_row(row_v, hist_v, cnt_v, kk)
        pltpu.sync_copy(row_v, out_hbm.at[pl.ds(base, _N)])


def kernel(scores, k):
    rows, cols = scores.shape
    kvec = jnp.full((_L,), k, jnp.int32)
    bits = lax.bitcast_convert_type(scores.reshape(-1), jnp.int32)
    out = _sc_topk(bits, kvec)
    return lax.bitcast_convert_type(out, jnp.float32).reshape(rows, cols)


# speedup vs baseline: 2.3513x; 2.0148x over previous
import functools

import jax
import jax.numpy as jnp
from jax import lax
from jax.experimental import pallas as pl
from jax.experimental.pallas import tpu as pltpu
from jax.experimental.pallas import tpu_sc as plsc

_L = 16
_NBINS = 256
_N = 32768
_ROWS = 64
_NW = 32
_ROWS_PER_W = _ROWS // _NW


_U = 8


def _zero_hist(hist_ref):
    z = jnp.zeros((_L,), jnp.int32)

    @plsc.parallel_loop(0, _NBINS * _L, _L, unroll=_U)
    def _(i):
        hist_ref[pl.ds(i, _L)] = z


def _hist_pass(row_ref, hist_ref, shift, prefix, lvl):
    ones = jnp.ones((_L,), jnp.int32)
    laneoff = lax.iota(jnp.int32, _L) * _NBINS

    @plsc.parallel_loop(0, _N, _L, unroll=_U)
    def _(i):
        b = row_ref[pl.ds(i, _L)] & jnp.int32(0x7FFFFFFF)
        dig = lax.shift_right_logical(b, shift) & jnp.int32(0xFF)
        idx = laneoff + dig
        if lvl == 0:
            plsc.addupdate_scatter(hist_ref, [idx], ones)
        else:
            m = lax.shift_right_logical(b, shift + 8) == prefix
            plsc.addupdate_scatter(hist_ref, [idx], ones, mask=m)


def _collapse(hist_ref, cnt_ref):
    @plsc.parallel_loop(0, _NBINS, _L, unroll=2)
    def _(g):
        acc = hist_ref[pl.ds(g, _L)]
        for lane in range(1, _L):
            acc = acc + hist_ref[pl.ds(lane * _NBINS + g, _L)]
        cnt_ref[pl.ds(g, _L)] = acc


def _gather16(x, idx):
    dn = lax.GatherDimensionNumbers(
        offset_dims=(), collapsed_slice_dims=(0,), start_index_map=(0,))
    return lax.gather(
        x, idx.reshape(_L, 1), dn, slice_sizes=(1,),
        mode=lax.GatherScatterMode.PROMISE_IN_BOUNDS)


def _cumsum16(x):
    lanes = lax.iota(jnp.int32, _L)
    s = x
    for d in (1, 2, 4, 8):
        sh = _gather16(s, jnp.maximum(lanes - d, 0))
        s = s + jnp.where(lanes >= d, sh, 0)
    return s


def _top_lane():
    return jnp.full((_L,), _L - 1, jnp.int32)


def _select_digit(cnt_ref, j):
    zero = jnp.zeros((_L,), jnp.int32)

    def body(i, carry):
        above, d_star, j_next, done = carry
        g = 15 - i
        h = cnt_ref[pl.ds(g * _L, _L)]
        cs = _cumsum16(h)
        gsum = _gather16(cs, _top_lane())
        s_vec = above + gsum - cs + h
        m = s_vec >= j
        pc = _gather16(_cumsum16(m.astype(jnp.int32)), _top_lane())
        hit = jnp.logical_and(pc > 0, done == 0)
        lanepos = jnp.maximum(pc - 1, 0)
        cs_at = _gather16(cs, lanepos)
        d_star = jnp.where(hit, g * _L + lanepos, d_star)
        j_next = jnp.where(hit, j - (above + gsum - cs_at), j_next)
        done = jnp.where(pc > 0, 1, done)
        above = jnp.where(done > 0, above, above + gsum)
        return above, d_star, j_next, done

    _, d_star, j_next, _ = lax.fori_loop(
        0, _NBINS // _L, body, (zero, zero, j, zero))
    return d_star, j_next


def _process_row(row_ref, hist_ref, cnt_ref, kk):
    prefix = jnp.zeros((_L,), jnp.int32)
    j = kk
    for lvl in range(4):
        shift = 24 - 8 * lvl
        _zero_hist(hist_ref)
        _hist_pass(row_ref, hist_ref, shift, prefix, lvl)
        _collapse(hist_ref, cnt_ref)
        d, j = _select_digit(cnt_ref, j)
        prefix = prefix * 256 + d
    thr = prefix

    @plsc.parallel_loop(0, _N, _L, unroll=_U)
    def _(i):
        v = row_ref[pl.ds(i, _L)]
        b = v & jnp.int32(0x7FFFFFFF)
        row_ref[pl.ds(i, _L)] = jnp.where(b >= thr, v, 0)


_MESH = plsc.VectorSubcoreMesh(core_axis_name="c", subcore_axis_name="s")


@functools.partial(
    pl.kernel,
    mesh=_MESH,
    compiler_params=pltpu.CompilerParams(needs_layout_passes=False),
    out_type=jax.ShapeDtypeStruct((_ROWS * _N,), jnp.int32),
    scratch_types=[
        pltpu.VMEM((_N,), jnp.int32),
        pltpu.VMEM((_NBINS * _L,), jnp.int32),
        pltpu.VMEM((_NBINS,), jnp.int32),
        pltpu.VMEM((_L,), jnp.int32),
    ],
)
def _sc_topk(scores_hbm, kvec_hbm, out_hbm, row_v, hist_v, cnt_v, k_v):
    wid = lax.axis_index("s") * 2 + lax.axis_index("c")
    pltpu.sync_copy(kvec_hbm, k_v)
    kk = k_v[...]
    for r in range(_ROWS_PER_W):
        row = wid * _ROWS_PER_W + r
        base = row * _N
        pltpu.sync_copy(scores_hbm.at[pl.ds(base, _N)], row_v)
        _process_row(row_v, hist_v, cnt_v, kk)
        pltpu.sync_copy(row_v, out_hbm.at[pl.ds(base, _N)])


def kernel(scores, k):
    rows, cols = scores.shape
    kvec = jnp.full((_L,), k, jnp.int32)
    bits = lax.bitcast_convert_type(scores.reshape(-1), jnp.int32)
    out = _sc_topk(bits, kvec)
    return lax.bitcast_convert_type(out, jnp.float32).reshape(rows, cols)


# speedup vs baseline: 2.9293x; 1.2458x over previous
import functools

import jax
import jax.numpy as jnp
from jax import lax
from jax.experimental import pallas as pl
from jax.experimental.pallas import tpu as pltpu
from jax.experimental.pallas import tpu_sc as plsc

_L = 16
_NBINS = 256
_N = 32768
_ROWS = 64
_NW = 32
_ROWS_PER_W = _ROWS // _NW


_U = 8


def _zero_hist(hist_ref):
    z = jnp.zeros((_L,), jnp.int32)

    @plsc.parallel_loop(0, _NBINS * _L, _L, unroll=_U)
    def _(i):
        hist_ref[pl.ds(i, _L)] = z


def _hist_pass(row_ref, hist_ref, shift, prefix, lvl):
    ones = jnp.ones((_L,), jnp.int32)
    laneoff = lax.iota(jnp.int32, _L) * _NBINS

    @plsc.parallel_loop(0, _N, _L, unroll=_U)
    def _(i):
        b = row_ref[pl.ds(i, _L)] & jnp.int32(0x7FFFFFFF)
        dig = lax.shift_right_logical(b, shift) & jnp.int32(0xFF)
        idx = laneoff + dig
        if lvl == 0:
            plsc.addupdate_scatter(hist_ref, [idx], ones)
        else:
            m = lax.shift_right_logical(b, shift + 8) == prefix
            plsc.addupdate_scatter(hist_ref, [idx], ones, mask=m)


def _collapse(hist_ref, cnt_ref):
    @plsc.parallel_loop(0, _NBINS, _L, unroll=2)
    def _(g):
        acc = hist_ref[pl.ds(g, _L)]
        for lane in range(1, _L):
            acc = acc + hist_ref[pl.ds(lane * _NBINS + g, _L)]
        cnt_ref[pl.ds(g, _L)] = acc


def _gather16(x, idx):
    dn = lax.GatherDimensionNumbers(
        offset_dims=(), collapsed_slice_dims=(0,), start_index_map=(0,))
    return lax.gather(
        x, idx.reshape(_L, 1), dn, slice_sizes=(1,),
        mode=lax.GatherScatterMode.PROMISE_IN_BOUNDS)


def _cumsum16(x):
    lanes = lax.iota(jnp.int32, _L)
    s = x
    for d in (1, 2, 4, 8):
        sh = _gather16(s, jnp.maximum(lanes - d, 0))
        s = s + jnp.where(lanes >= d, sh, 0)
    return s


def _top_lane():
    return jnp.full((_L,), _L - 1, jnp.int32)


def _select_digit(cnt_ref, j):
    zero = jnp.zeros((_L,), jnp.int32)

    def body(i, carry):
        above, d_star, j_next, done = carry
        g = 15 - i
        h = cnt_ref[pl.ds(g * _L, _L)]
        cs = _cumsum16(h)
        gsum = _gather16(cs, _top_lane())
        s_vec = above + gsum - cs + h
        m = s_vec >= j
        pc = _gather16(_cumsum16(m.astype(jnp.int32)), _top_lane())
        hit = jnp.logical_and(pc > 0, done == 0)
        lanepos = jnp.maximum(pc - 1, 0)
        cs_at = _gather16(cs, lanepos)
        d_star = jnp.where(hit, g * _L + lanepos, d_star)
        j_next = jnp.where(hit, j - (above + gsum - cs_at), j_next)
        done = jnp.where(pc > 0, 1, done)
        above = jnp.where(done > 0, above, above + gsum)
        return above, d_star, j_next, done

    _, d_star, j_next, _ = lax.fori_loop(
        0, _NBINS // _L, body, (zero, zero, j, zero))
    return d_star, j_next


def _process_row(row_ref, hist_ref, cnt_ref, kk):
    prefix = jnp.zeros((_L,), jnp.int32)
    j = kk
    for lvl in range(4):
        shift = 24 - 8 * lvl
        _zero_hist(hist_ref)
        _hist_pass(row_ref, hist_ref, shift, prefix, lvl)
        _collapse(hist_ref, cnt_ref)
        d, j = _select_digit(cnt_ref, j)
        prefix = prefix * 256 + d
    thr = prefix

    @plsc.parallel_loop(0, _N, _L, unroll=_U)
    def _(i):
        v = row_ref[pl.ds(i, _L)]
        b = v & jnp.int32(0x7FFFFFFF)
        row_ref[pl.ds(i, _L)] = jnp.where(b >= thr, v, 0)


_MESH = plsc.VectorSubcoreMesh(core_axis_name="c", subcore_axis_name="s")


@functools.partial(
    pl.kernel,
    mesh=_MESH,
    compiler_params=pltpu.CompilerParams(needs_layout_passes=False),
    out_type=jax.ShapeDtypeStruct((_ROWS, _N), jnp.int32),
    scratch_types=[
        pltpu.VMEM((_N,), jnp.int32),
        pltpu.VMEM((_NBINS * _L,), jnp.int32),
        pltpu.VMEM((_NBINS,), jnp.int32),
        pltpu.VMEM((_L,), jnp.int32),
    ],
)
def _sc_topk(scores_hbm, kvec_hbm, out_hbm, row_v, hist_v, cnt_v, k_v):
    wid = lax.axis_index("s") * 2 + lax.axis_index("c")
    pltpu.sync_copy(kvec_hbm, k_v)
    kk = k_v[...]
    for r in range(_ROWS_PER_W):
        row = wid * _ROWS_PER_W + r
        pltpu.sync_copy(scores_hbm.at[row], row_v)
        _process_row(row_v, hist_v, cnt_v, kk)
        pltpu.sync_copy(row_v, out_hbm.at[row])


def kernel(scores, k):
    kvec = jnp.full((_L,), k, jnp.int32)
    bits = lax.bitcast_convert_type(scores, jnp.int32)
    out = _sc_topk(bits, kvec)
    return lax.bitcast_convert_type(out, jnp.float32)


# speedup vs baseline: 3.0268x; 1.0333x over previous
import functools

import jax
import jax.numpy as jnp
from jax import lax
from jax.experimental import pallas as pl
from jax.experimental.pallas import tpu as pltpu
from jax.experimental.pallas import tpu_sc as plsc

_L = 16
_NBINS = 256
_N = 32768
_ROWS = 64
_NW = 32
_ROWS_PER_W = _ROWS // _NW


_U = 8


def _zero_hist(hist_ref):
    z = jnp.zeros((_L,), jnp.int32)

    @plsc.parallel_loop(0, _NBINS, _L, unroll=_U)
    def _(i):
        hist_ref[pl.ds(i, _L)] = z


def _hist_pass(row_ref, hist_ref, shift, prefix, lvl):
    ones = jnp.ones((_L,), jnp.int32)

    @plsc.parallel_loop(0, _N, _L, unroll=_U)
    def _(i):
        b = row_ref[pl.ds(i, _L)] & jnp.int32(0x7FFFFFFF)
        dig = lax.shift_right_logical(b, shift) & jnp.int32(0xFF)
        if lvl == 0:
            plsc.addupdate_scatter(hist_ref, [dig], ones)
        else:
            m = lax.shift_right_logical(b, shift + 8) == prefix
            plsc.addupdate_scatter(hist_ref, [dig], ones, mask=m)


def _gather16(x, idx):
    dn = lax.GatherDimensionNumbers(
        offset_dims=(), collapsed_slice_dims=(0,), start_index_map=(0,))
    return lax.gather(
        x, idx.reshape(_L, 1), dn, slice_sizes=(1,),
        mode=lax.GatherScatterMode.PROMISE_IN_BOUNDS)


def _cumsum16(x):
    lanes = lax.iota(jnp.int32, _L)
    s = x
    for d in (1, 2, 4, 8):
        sh = _gather16(s, jnp.maximum(lanes - d, 0))
        s = s + jnp.where(lanes >= d, sh, 0)
    return s


def _top_lane():
    return jnp.full((_L,), _L - 1, jnp.int32)


def _select_digit(cnt_ref, j):
    zero = jnp.zeros((_L,), jnp.int32)

    def body(i, carry):
        above, d_star, j_next, done = carry
        g = 15 - i
        h = cnt_ref[pl.ds(g * _L, _L)]
        cs = _cumsum16(h)
        gsum = _gather16(cs, _top_lane())
        s_vec = above + gsum - cs + h
        m = s_vec >= j
        pc = _gather16(_cumsum16(m.astype(jnp.int32)), _top_lane())
        hit = jnp.logical_and(pc > 0, done == 0)
        lanepos = jnp.maximum(pc - 1, 0)
        cs_at = _gather16(cs, lanepos)
        d_star = jnp.where(hit, g * _L + lanepos, d_star)
        j_next = jnp.where(hit, j - (above + gsum - cs_at), j_next)
        done = jnp.where(pc > 0, 1, done)
        above = jnp.where(done > 0, above, above + gsum)
        return above, d_star, j_next, done

    _, d_star, j_next, _ = lax.fori_loop(
        0, _NBINS // _L, body, (zero, zero, j, zero))
    return d_star, j_next


def _process_row(row_ref, hist_ref, kk):
    prefix = jnp.zeros((_L,), jnp.int32)
    j = kk
    for lvl in range(4):
        shift = 24 - 8 * lvl
        _zero_hist(hist_ref)
        _hist_pass(row_ref, hist_ref, shift, prefix, lvl)
        d, j = _select_digit(hist_ref, j)
        prefix = prefix * 256 + d
    thr = prefix

    @plsc.parallel_loop(0, _N, _L, unroll=_U)
    def _(i):
        v = row_ref[pl.ds(i, _L)]
        b = v & jnp.int32(0x7FFFFFFF)
        row_ref[pl.ds(i, _L)] = jnp.where(b >= thr, v, 0)


_MESH = plsc.VectorSubcoreMesh(core_axis_name="c", subcore_axis_name="s")


@functools.partial(
    pl.kernel,
    mesh=_MESH,
    compiler_params=pltpu.CompilerParams(needs_layout_passes=False),
    out_type=jax.ShapeDtypeStruct((_ROWS, _N), jnp.int32),
    scratch_types=[
        pltpu.VMEM((_N,), jnp.int32),
        pltpu.VMEM((_NBINS,), jnp.int32),
        pltpu.VMEM((_L,), jnp.int32),
    ],
)
def _sc_topk(scores_hbm, kvec_hbm, out_hbm, row_v, hist_v, k_v):
    wid = lax.axis_index("s") * 2 + lax.axis_index("c")
    pltpu.sync_copy(kvec_hbm, k_v)
    kk = k_v[...]
    for r in range(_ROWS_PER_W):
        row = wid * _ROWS_PER_W + r
        pltpu.sync_copy(scores_hbm.at[row], row_v)
        _process_row(row_v, hist_v, kk)
        pltpu.sync_copy(row_v, out_hbm.at[row])


def kernel(scores, k):
    kvec = jnp.full((_L,), k, jnp.int32)
    bits = lax.bitcast_convert_type(scores, jnp.int32)
    out = _sc_topk(bits, kvec)
    return lax.bitcast_convert_type(out, jnp.float32)
